# Initial kernel scaffold; baseline (speedup 1.0000x reference)
import jax
import jax.numpy as jnp
from jax.experimental import pallas as pl


def kernel(x, edge_index, edge_attr, Wz, bz, Wr, br, Wh, bh, Wlz, blz, Wlr, blr, Wlh, blh, attention):
    raise NotImplementedError("write your pallas kernel here")


# speedup vs baseline: 13.5671x; 13.5671x over previous
import functools

import jax
import jax.numpy as jnp
from jax import lax
from jax.experimental import pallas as pl
from jax.experimental.pallas import tpu as pltpu
from jax.experimental.pallas import tpu_sc as plsc


_LANES = 16
_W = 128


def _make_sc_deg(N, E):
  assert E % _W == 0 and N % 8 == 0
  nwin = E // _W
  info = plsc.get_sparse_core_info()
  nsub = info.num_subcores
  npad = ((N + 2047) // 2048) * 2048
  zchunk = 2048
  mesh = plsc.VectorSubcoreMesh(core_axis_name="c", subcore_axis_name="s")

  @functools.partial(
      pl.kernel,
      mesh=mesh,
      out_type=jax.ShapeDtypeStruct((npad,), jnp.float32),
      scratch_types=[
          pltpu.VMEM_SHARED((npad,), jnp.float32),
          pltpu.VMEM((zchunk,), jnp.float32),
          pltpu.VMEM((_W,), jnp.int32),
          pltpu.VMEM((_W,), jnp.float32),
      ],
  )
  def deg_kernel(dst_hbm, ew_hbm, out_hbm, deg_sh, z1, dst_v, ew_v):
    c = lax.axis_index("c")
    s = lax.axis_index("s")
    zero16 = jnp.zeros((_LANES,), jnp.float32)

    def _fill1(i, _):
      z1[pl.ds(i * _LANES, _LANES)] = zero16
      return 0
    lax.fori_loop(0, zchunk // _LANES, _fill1, 0)

    @pl.when(s == 0)
    def _zero_deg():
      for k in range(npad // zchunk):
        pltpu.sync_copy(z1, deg_sh.at[pl.ds(k * zchunk, zchunk)])
    plsc.subcore_barrier()

    n_iters = (nwin + nsub - 1) // nsub

    def _deg_body(i, _):
      w = i * nsub + s
      @pl.when(w < nwin)
      def _():
        pltpu.sync_copy(dst_hbm.at[pl.ds(w * _W, _W)], dst_v)
        pltpu.sync_copy(ew_hbm.at[pl.ds(w * _W, _W)], ew_v)
        pltpu.sync_copy(ew_v, deg_sh.at[dst_v], add=True)
      return 0
    lax.fori_loop(0, n_iters, _deg_body, 0)
    plsc.subcore_barrier()

    chunk = npad // nsub
    @pl.when(c == 0)
    def _copy_out():
      pltpu.sync_copy(deg_sh.at[pl.ds(s * chunk, chunk)],
                      out_hbm.at[pl.ds(s * chunk, chunk)])

  return deg_kernel


def _make_sc_spmm(N, E, F, P):
  assert F % _LANES == 0 and E % _W == 0 and N % 8 == 0
  nwin = E // _W
  info = plsc.get_sparse_core_info()
  ncores, nsub = info.num_cores, info.num_subcores
  assert P % ncores == 0
  p_per_core = P // ncores
  rpt = (N // nsub) // 8 * 8
  rem = N - nsub * rpt
  mesh = plsc.VectorSubcoreMesh(core_axis_name="c", subcore_axis_name="s")

  @functools.partial(
      pl.kernel,
      mesh=mesh,
      out_type=jax.ShapeDtypeStruct((P * N, F), jnp.float32),
      scratch_types=[
          pltpu.VMEM_SHARED((N, F), jnp.float32),
          pltpu.VMEM((208, F), jnp.float32),
          pltpu.VMEM((_W,), jnp.int32),
          pltpu.VMEM((_W,), jnp.int32),
          pltpu.VMEM((_W,), jnp.float32),
          pltpu.VMEM((_W,), jnp.int32),
          pltpu.VMEM((_W, F), jnp.float32),
          pltpu.SemaphoreType.DMA,
      ],
  )
  def spmm(x_hbm, src_hbm, dst_hbm, ew_hbm, out_hbm,
           y_sh, z2, src_v, dst_v, ew_v, gidx_v, rows_v, sem):
    c = lax.axis_index("c")
    s = lax.axis_index("s")
    zero16 = jnp.zeros((_LANES,), jnp.float32)

    def _fill2(i, _):
      r = i // (F // _LANES)
      col = (i % (F // _LANES)) * _LANES
      z2[r, pl.ds(col, _LANES)] = zero16
      return 0
    lax.fori_loop(0, 208 * (F // _LANES), _fill2, 0)

    n_iters = (nwin + nsub - 1) // nsub

    for k in range(p_per_core):
      p = c * p_per_core + k
      r0 = s * rpt
      for q in range(rpt // 208):
        pltpu.sync_copy(z2, y_sh.at[pl.ds(r0 + q * 208, 208)])
      @pl.when(s == nsub - 1)
      def _zero_rem():
        pltpu.sync_copy(z2.at[pl.ds(0, rem)], y_sh.at[pl.ds(nsub * rpt, rem)])
      plsc.subcore_barrier()

      def _win_body(i, _):
        w = i * nsub + s
        @pl.when(w < nwin)
        def _():
          e0 = w * _W
          pltpu.sync_copy(src_hbm.at[pl.ds(e0, _W)], src_v)
          pltpu.sync_copy(dst_hbm.at[pl.ds(e0, _W)], dst_v)
          pltpu.sync_copy(ew_hbm.at[pl.ds(e0, _W)], ew_v)
          base = p * N
          for j in range(_W // _LANES):
            sl = pl.ds(j * _LANES, _LANES)
            gidx_v[sl] = src_v[sl] + base
          pltpu.async_copy(x_hbm.at[gidx_v], rows_v, sem).wait()

          def _scale_rows(g, _):
            nv = ew_v[pl.ds(g * _LANES, _LANES)]
            for l in range(_LANES):
              r = g * _LANES + l
              nb = jnp.full((_LANES,), nv[l])
              for j in range(F // _LANES):
                sl = pl.ds(j * _LANES, _LANES)
                rows_v[r, sl] = rows_v[r, sl] * nb
            return 0
          lax.fori_loop(0, _W // _LANES, _scale_rows, 0)
          pltpu.sync_copy(rows_v, y_sh.at[dst_v], add=True)
        return 0
      lax.fori_loop(0, n_iters, _win_body, 0)
      plsc.subcore_barrier()
      pltpu.sync_copy(y_sh.at[pl.ds(r0, rpt)],
                      out_hbm.at[pl.ds(p * N + r0, rpt)])
      @pl.when(s == nsub - 1)
      def _copy_rem():
        pltpu.sync_copy(y_sh.at[pl.ds(nsub * rpt, rem)],
                        out_hbm.at[pl.ds(p * N + nsub * rpt, rem)])
      plsc.subcore_barrier()

  return spmm


def _prep_body(deg_ref, x_ref, xs_ref, dinv_ref):
  deg = deg_ref[...]
  dinv = jnp.where(deg > 0.0, lax.rsqrt(jnp.where(deg > 0.0, deg, 1.0)), 0.0)
  xs_ref[...] = x_ref[...] * dinv
  dinv_ref[...] = dinv


def _gate_body(y_ref, dinv_ref, wz_ref, wh_ref, wlz_ref, wlh_ref,
               bz_ref, bh_ref, blz_ref, blh_ref, att_ref, out_ref):
  P = y_ref.shape[0]
  att = att_ref[...]
  att = att - jnp.max(att, axis=1, keepdims=True)
  ea = jnp.exp(att)
  probs = ea / jnp.sum(ea, axis=1, keepdims=True)
  az = jnp.dot(wz_ref[...], wlz_ref[...], preferred_element_type=jnp.float32)
  ah = jnp.dot(wh_ref[...], wlh_ref[...], preferred_element_type=jnp.float32)
  cz = jnp.dot(bz_ref[...], wlz_ref[...],
               preferred_element_type=jnp.float32) + blz_ref[...]
  ch = jnp.dot(bh_ref[...], wlh_ref[...],
               preferred_element_type=jnp.float32) + blh_ref[...]
  dinv = dinv_ref[...]
  acc = jnp.zeros(out_ref.shape, jnp.float32)
  for p in range(P):
    yp = y_ref[p] * dinv
    z = jax.nn.sigmoid(jnp.dot(yp, az, preferred_element_type=jnp.float32)
                       + cz)
    t = jnp.tanh(jnp.dot(yp, ah, preferred_element_type=jnp.float32) + ch)
    acc = acc + probs[0, p] * (1.0 - z) * t
  out_ref[...] = acc


def kernel(x, edge_index, edge_attr, Wz, bz, Wr, br, Wh, bh,
           Wlz, blz, Wlr, blr, Wlh, blh, attention):
  N, F, P = x.shape
  E = edge_index.shape[1]
  OUT = Wz.shape[1]
  del Wr, br, Wlr, blr

  xT = jnp.transpose(x, (2, 0, 1)).reshape(P * N, F)
  src = edge_index[0]
  dst = edge_index[1]

  deg = _make_sc_deg(N, E)(dst, edge_attr)
  deg2 = deg[:N].reshape(N, 1)

  R1 = 400
  xs, dinv = pl.pallas_call(
      _prep_body,
      grid=(P, N // R1),
      in_specs=[
          pl.BlockSpec((R1, 1), lambda p, i: (i, 0)),
          pl.BlockSpec((R1, F), lambda p, i, n_blocks=N // R1:
                       (p * n_blocks + i, 0)),
      ],
      out_specs=[
          pl.BlockSpec((R1, F), lambda p, i, n_blocks=N // R1:
                       (p * n_blocks + i, 0)),
          pl.BlockSpec((R1, 1), lambda p, i: (i, 0)),
      ],
      out_shape=[
          jax.ShapeDtypeStruct((P * N, F), jnp.float32),
          jax.ShapeDtypeStruct((N, 1), jnp.float32),
      ],
  )(deg2, xT)

  y = _make_sc_spmm(N, E, F, P)(xs, src, dst, edge_attr)
  y = y.reshape(P, N, F)

  R = 1000
  out = pl.pallas_call(
      _gate_body,
      grid=(N // R,),
      in_specs=[
          pl.BlockSpec((P, R, F), lambda i: (0, i, 0)),
          pl.BlockSpec((R, 1), lambda i: (i, 0)),
          pl.BlockSpec((F, OUT), lambda i: (0, 0)),
          pl.BlockSpec((F, OUT), lambda i: (0, 0)),
          pl.BlockSpec((OUT, OUT), lambda i: (0, 0)),
          pl.BlockSpec((OUT, OUT), lambda i: (0, 0)),
          pl.BlockSpec((1, OUT), lambda i: (0, 0)),
          pl.BlockSpec((1, OUT), lambda i: (0, 0)),
          pl.BlockSpec((1, OUT), lambda i: (0, 0)),
          pl.BlockSpec((1, OUT), lambda i: (0, 0)),
          pl.BlockSpec((1, P), lambda i: (0, 0)),
      ],
      out_specs=pl.BlockSpec((R, OUT), lambda i: (i, 0)),
      out_shape=jax.ShapeDtypeStruct((N, OUT), jnp.float32),
  )(y, dinv, Wz, Wh, Wlz, Wlh,
    bz.reshape(1, OUT), bh.reshape(1, OUT),
    blz.reshape(1, OUT), blh.reshape(1, OUT),
    attention.reshape(1, P))
  return out


# speedup vs baseline: 23.7305x; 1.7491x over previous
import functools

import jax
import jax.numpy as jnp
from jax import lax
from jax.experimental import pallas as pl
from jax.experimental.pallas import tpu as pltpu
from jax.experimental.pallas import tpu_sc as plsc


_LANES = 16
_W = 128


def _make_sc_deg(N, E):
  assert E % _W == 0 and N % 8 == 0
  nwin = E // _W
  info = plsc.get_sparse_core_info()
  nsub = info.num_subcores
  npad = ((N + 2047) // 2048) * 2048
  zchunk = 2048
  mesh = plsc.VectorSubcoreMesh(core_axis_name="c", subcore_axis_name="s")

  @functools.partial(
      pl.kernel,
      mesh=mesh,
      out_type=jax.ShapeDtypeStruct((npad,), jnp.float32),
      scratch_types=[
          pltpu.VMEM_SHARED((npad,), jnp.float32),
          pltpu.VMEM((zchunk,), jnp.float32),
          pltpu.VMEM((_W,), jnp.int32),
          pltpu.VMEM((_W,), jnp.float32),
      ],
  )
  def deg_kernel(dst_hbm, ew_hbm, out_hbm, deg_sh, z1, dst_v, ew_v):
    c = lax.axis_index("c")
    s = lax.axis_index("s")
    zero16 = jnp.zeros((_LANES,), jnp.float32)

    def _fill1(i, _):
      z1[pl.ds(i * _LANES, _LANES)] = zero16
      return 0
    lax.fori_loop(0, zchunk // _LANES, _fill1, 0)

    @pl.when(s == 0)
    def _zero_deg():
      for k in range(npad // zchunk):
        pltpu.sync_copy(z1, deg_sh.at[pl.ds(k * zchunk, zchunk)])
    plsc.subcore_barrier()

    n_iters = (nwin + nsub - 1) // nsub

    def _deg_body(i, _):
      w = i * nsub + s
      @pl.when(w < nwin)
      def _():
        pltpu.sync_copy(dst_hbm.at[pl.ds(w * _W, _W)], dst_v)
        pltpu.sync_copy(ew_hbm.at[pl.ds(w * _W, _W)], ew_v)
        pltpu.sync_copy(ew_v, deg_sh.at[dst_v], add=True)
      return 0
    lax.fori_loop(0, n_iters, _deg_body, 0)
    plsc.subcore_barrier()

    chunk = npad // nsub
    @pl.when(c == 0)
    def _copy_out():
      pltpu.sync_copy(deg_sh.at[pl.ds(s * chunk, chunk)],
                      out_hbm.at[pl.ds(s * chunk, chunk)])

  return deg_kernel


def _make_sc_spmm(N, E, F, P):
  assert F % _LANES == 0 and E % _W == 0 and N % 8 == 0
  nwin = E // _W
  info = plsc.get_sparse_core_info()
  ncores, nsub = info.num_cores, info.num_subcores
  assert P % ncores == 0
  p_per_core = P // ncores
  rpt = (N // nsub) // 8 * 8
  rem = N - nsub * rpt
  mesh = plsc.VectorSubcoreMesh(core_axis_name="c", subcore_axis_name="s")

  @functools.partial(
      pl.kernel,
      mesh=mesh,
      out_type=jax.ShapeDtypeStruct((P * N, F), jnp.float32),
      scratch_types=[
          pltpu.VMEM_SHARED((N, F), jnp.float32),
          pltpu.VMEM((48, F), jnp.float32),
          pltpu.VMEM((_W,), jnp.int32),
          pltpu.VMEM((_W,), jnp.int32),
          pltpu.VMEM((_W,), jnp.int32),
          pltpu.VMEM((_W,), jnp.int32),
          pltpu.VMEM((_W,), jnp.float32),
          pltpu.VMEM((_W,), jnp.float32),
          pltpu.VMEM((_W,), jnp.int32),
          pltpu.VMEM((_W,), jnp.int32),
          pltpu.VMEM((_W, F), jnp.float32),
          pltpu.VMEM((_W, F), jnp.float32),
          pltpu.SemaphoreType.DMA,
          pltpu.SemaphoreType.DMA,
          pltpu.SemaphoreType.DMA,
          pltpu.SemaphoreType.DMA,
          pltpu.SemaphoreType.DMA,
          pltpu.SemaphoreType.DMA,
      ],
  )
  def spmm(x_hbm, src_hbm, dst_hbm, ew_hbm, out_hbm,
           y_sh, z2, src0, src1, dst0, dst1, ew0, ew1, gidx0, gidx1,
           rows0, rows1, esem0, esem1, gsem0, gsem1, ssem0, ssem1):
    srcb = (src0, src1)
    dstb = (dst0, dst1)
    ewb = (ew0, ew1)
    gidxb = (gidx0, gidx1)
    rowsb = (rows0, rows1)
    esem = (esem0, esem1)
    gsem = (gsem0, gsem1)
    ssem = (ssem0, ssem1)
    c = lax.axis_index("c")
    s = lax.axis_index("s")
    zero16 = jnp.zeros((_LANES,), jnp.float32)

    def _fill2(i, _):
      r = i // (F // _LANES)
      col = (i % (F // _LANES)) * _LANES
      z2[r, pl.ds(col, _LANES)] = zero16
      return 0
    lax.fori_loop(0, 48 * (F // _LANES), _fill2, 0)

    n_iters = (nwin + nsub - 1) // nsub

    for k in range(p_per_core):
      p = c * p_per_core + k
      r0 = s * rpt
      for q in range(rpt // 48):
        pltpu.sync_copy(z2, y_sh.at[pl.ds(r0 + q * 48, 48)])
      @pl.when(s == nsub - 1)
      def _zero_rem():
        pltpu.sync_copy(z2.at[pl.ds(0, rem)], y_sh.at[pl.ds(nsub * rpt, rem)])
      plsc.subcore_barrier()

      base = p * N

      def _guard(it):
        return jnp.logical_and(it >= 0, it * nsub + s < nwin)

      def _stage_launch(it, b):
        @pl.when(_guard(it))
        def _():
          e0 = (it * nsub + s) * _W
          c1 = pltpu.async_copy(src_hbm.at[pl.ds(e0, _W)], srcb[b], esem[b])
          c2 = pltpu.async_copy(dst_hbm.at[pl.ds(e0, _W)], dstb[b], esem[b])
          c3 = pltpu.async_copy(ew_hbm.at[pl.ds(e0, _W)], ewb[b], esem[b])
          c1.wait()
          c2.wait()
          c3.wait()
          for j in range(_W // _LANES):
            sl = pl.ds(j * _LANES, _LANES)
            gidxb[b][sl] = srcb[b][sl] + base
          pltpu.async_copy(x_hbm.at[gidxb[b]], rowsb[b], gsem[b])

      def _stage_process(it, b):
        @pl.when(_guard(it))
        def _():
          pltpu.make_async_copy(x_hbm.at[gidxb[b]], rowsb[b], gsem[b]).wait()

          def _scale(g, _):
            nv = ewb[b][pl.ds(g * _LANES, _LANES)]
            for l in range(_LANES):
              r = g * _LANES + l
              nb = jnp.full((_LANES,), nv[l])
              for j in range(F // _LANES):
                sl = pl.ds(j * _LANES, _LANES)
                rowsb[b][r, sl] = rowsb[b][r, sl] * nb
            return 0
          lax.fori_loop(0, _W // _LANES, _scale, 0)
          pltpu.async_copy(rowsb[b], y_sh.at[dstb[b]], ssem[b], add=True)

      def _stage_drain(it, b):
        @pl.when(_guard(it))
        def _():
          pltpu.make_async_copy(rowsb[b], y_sh.at[dstb[b]], ssem[b]).wait()

      def _pair_body(i, _):
        for b2 in range(2):
          it = i * 2 + b2
          _stage_drain(it - 2, b2)
          _stage_launch(it, b2)
          _stage_process(it - 1, 1 - b2)
        return 0
      lax.fori_loop(0, (n_iters + 4) // 2, _pair_body, 0)
      plsc.subcore_barrier()
      pltpu.sync_copy(y_sh.at[pl.ds(r0, rpt)],
                      out_hbm.at[pl.ds(p * N + r0, rpt)])
      @pl.when(s == nsub - 1)
      def _copy_rem():
        pltpu.sync_copy(y_sh.at[pl.ds(nsub * rpt, rem)],
                        out_hbm.at[pl.ds(p * N + nsub * rpt, rem)])
      plsc.subcore_barrier()

  return spmm


def _prep_body(deg_ref, x_ref, xs_ref, dinv_ref):
  deg = deg_ref[...]
  dinv = jnp.where(deg > 0.0, lax.rsqrt(jnp.where(deg > 0.0, deg, 1.0)), 0.0)
  xs_ref[...] = x_ref[...] * dinv
  dinv_ref[...] = dinv


def _gate_body(y_ref, dinv_ref, wz_ref, wh_ref, wlz_ref, wlh_ref,
               bz_ref, bh_ref, blz_ref, blh_ref, att_ref, out_ref):
  P = y_ref.shape[0]
  att = att_ref[...]
  att = att - jnp.max(att, axis=1, keepdims=True)
  ea = jnp.exp(att)
  probs = ea / jnp.sum(ea, axis=1, keepdims=True)
  az = jnp.dot(wz_ref[...], wlz_ref[...], preferred_element_type=jnp.float32)
  ah = jnp.dot(wh_ref[...], wlh_ref[...], preferred_element_type=jnp.float32)
  cz = jnp.dot(bz_ref[...], wlz_ref[...],
               preferred_element_type=jnp.float32) + blz_ref[...]
  ch = jnp.dot(bh_ref[...], wlh_ref[...],
               preferred_element_type=jnp.float32) + blh_ref[...]
  dinv = dinv_ref[...]
  acc = jnp.zeros(out_ref.shape, jnp.float32)
  for p in range(P):
    yp = y_ref[p] * dinv
    z = jax.nn.sigmoid(jnp.dot(yp, az, preferred_element_type=jnp.float32)
                       + cz)
    t = jnp.tanh(jnp.dot(yp, ah, preferred_element_type=jnp.float32) + ch)
    acc = acc + probs[0, p] * (1.0 - z) * t
  out_ref[...] = acc


def kernel(x, edge_index, edge_attr, Wz, bz, Wr, br, Wh, bh,
           Wlz, blz, Wlr, blr, Wlh, blh, attention):
  N, F, P = x.shape
  E = edge_index.shape[1]
  OUT = Wz.shape[1]
  del Wr, br, Wlr, blr

  xT = jnp.transpose(x, (2, 0, 1)).reshape(P * N, F)
  src = edge_index[0]
  dst = edge_index[1]

  deg = _make_sc_deg(N, E)(dst, edge_attr)
  deg2 = deg[:N].reshape(N, 1)

  R1 = 400
  xs, dinv = pl.pallas_call(
      _prep_body,
      grid=(P, N // R1),
      in_specs=[
          pl.BlockSpec((R1, 1), lambda p, i: (i, 0)),
          pl.BlockSpec((R1, F), lambda p, i, n_blocks=N // R1:
                       (p * n_blocks + i, 0)),
      ],
      out_specs=[
          pl.BlockSpec((R1, F), lambda p, i, n_blocks=N // R1:
                       (p * n_blocks + i, 0)),
          pl.BlockSpec((R1, 1), lambda p, i: (i, 0)),
      ],
      out_shape=[
          jax.ShapeDtypeStruct((P * N, F), jnp.float32),
          jax.ShapeDtypeStruct((N, 1), jnp.float32),
      ],
  )(deg2, xT)

  y = _make_sc_spmm(N, E, F, P)(xs, src, dst, edge_attr)
  y = y.reshape(P, N, F)

  R = 1000
  out = pl.pallas_call(
      _gate_body,
      grid=(N // R,),
      in_specs=[
          pl.BlockSpec((P, R, F), lambda i: (0, i, 0)),
          pl.BlockSpec((R, 1), lambda i: (i, 0)),
          pl.BlockSpec((F, OUT), lambda i: (0, 0)),
          pl.BlockSpec((F, OUT), lambda i: (0, 0)),
          pl.BlockSpec((OUT, OUT), lambda i: (0, 0)),
          pl.BlockSpec((OUT, OUT), lambda i: (0, 0)),
          pl.BlockSpec((1, OUT), lambda i: (0, 0)),
          pl.BlockSpec((1, OUT), lambda i: (0, 0)),
          pl.BlockSpec((1, OUT), lambda i: (0, 0)),
          pl.BlockSpec((1, OUT), lambda i: (0, 0)),
          pl.BlockSpec((1, P), lambda i: (0, 0)),
      ],
      out_specs=pl.BlockSpec((R, OUT), lambda i: (i, 0)),
      out_shape=jax.ShapeDtypeStruct((N, OUT), jnp.float32),
  )(y, dinv, Wz, Wh, Wlz, Wlh,
    bz.reshape(1, OUT), bh.reshape(1, OUT),
    blz.reshape(1, OUT), blh.reshape(1, OUT),
    attention.reshape(1, P))
  return out


# speedup vs baseline: 32.3823x; 1.3646x over previous
import functools

import jax
import jax.numpy as jnp
from jax import lax
from jax.experimental import pallas as pl
from jax.experimental.pallas import tpu as pltpu
from jax.experimental.pallas import tpu_sc as plsc


_LANES = 16
_W = 128


def _make_sc_deg(N, E):
  assert E % _W == 0 and N % 8 == 0
  nwin = E // _W
  info = plsc.get_sparse_core_info()
  nsub = info.num_subcores
  npad = ((N + 2047) // 2048) * 2048
  zchunk = 2048
  mesh = plsc.VectorSubcoreMesh(core_axis_name="c", subcore_axis_name="s")

  @functools.partial(
      pl.kernel,
      mesh=mesh,
      out_type=jax.ShapeDtypeStruct((npad,), jnp.float32),
      scratch_types=(
          [pltpu.VMEM_SHARED((npad,), jnp.float32)]
          + [pltpu.VMEM((zchunk,), jnp.float32)]
          + [pltpu.VMEM((_W,), jnp.int32) for _ in range(4)]
          + [pltpu.VMEM((_W,), jnp.float32) for _ in range(4)]
          + [pltpu.SemaphoreType.DMA for _ in range(4)]
          + [pltpu.SemaphoreType.DMA for _ in range(2)]
      ),
  )
  def deg_kernel(dst_hbm, ew_hbm, out_hbm, deg_sh, z1,
                 d0, d1, d2, d3, e0, e1, e2, e3,
                 es0, es1, es2, es3, ss0, ss1):
    dstb = (d0, d1, d2, d3)
    ewb = (e0, e1, e2, e3)
    esem = (es0, es1, es2, es3)
    ssem = (ss0, ss1)
    c = lax.axis_index("c")
    s = lax.axis_index("s")
    zero16 = jnp.zeros((_LANES,), jnp.float32)

    def _fill1(i, _):
      z1[pl.ds(i * _LANES, _LANES)] = zero16
      return 0
    lax.fori_loop(0, zchunk // _LANES, _fill1, 0)

    @pl.when(s == 0)
    def _zero_deg():
      for k in range(npad // zchunk):
        pltpu.sync_copy(z1, deg_sh.at[pl.ds(k * zchunk, zchunk)])
    plsc.subcore_barrier()

    n_iters = (nwin + nsub - 1) // nsub

    def _guard(it):
      return jnp.logical_and(it >= 0, it * nsub + s < nwin)

    def _edges(it, b4):
      @pl.when(_guard(it))
      def _():
        e_off = (it * nsub + s) * _W
        pltpu.async_copy(dst_hbm.at[pl.ds(e_off, _W)], dstb[b4], esem[b4])
        pltpu.async_copy(ew_hbm.at[pl.ds(e_off, _W)], ewb[b4], esem[b4])

    def _launch(it, b4, b2):
      @pl.when(_guard(it))
      def _():
        e_off = (it * nsub + s) * _W
        pltpu.make_async_copy(dst_hbm.at[pl.ds(e_off, _W)], dstb[b4],
                              esem[b4]).wait()
        pltpu.make_async_copy(ew_hbm.at[pl.ds(e_off, _W)], ewb[b4],
                              esem[b4]).wait()
        pltpu.async_copy(ewb[b4], deg_sh.at[dstb[b4]], ssem[b2], add=True)

    def _drain(it, b4, b2):
      @pl.when(_guard(it))
      def _():
        pltpu.make_async_copy(ewb[b4], deg_sh.at[dstb[b4]], ssem[b2]).wait()

    _edges(0, 0)

    def _quad_body(i, _):
      for q in range(4):
        it = i * 4 + q
        _drain(it - 2, (q - 2) % 4, q % 2)
        _edges(it + 1, (q + 1) % 4)
        _launch(it, q, q % 2)
      return 0
    lax.fori_loop(0, (n_iters + 4) // 4 + 1, _quad_body, 0)
    plsc.subcore_barrier()

    chunk = npad // nsub
    @pl.when(c == 0)
    def _copy_out():
      pltpu.sync_copy(deg_sh.at[pl.ds(s * chunk, chunk)],
                      out_hbm.at[pl.ds(s * chunk, chunk)])

  return deg_kernel


def _make_sc_spmm(N, E, F, P):
  assert F % _LANES == 0 and E % _W == 0 and N % 8 == 0
  nwin = E // _W
  info = plsc.get_sparse_core_info()
  ncores, nsub = info.num_cores, info.num_subcores
  assert P % ncores == 0
  p_per_core = P // ncores
  rpt = (N // nsub) // 8 * 8
  rem = N - nsub * rpt
  mesh = plsc.VectorSubcoreMesh(core_axis_name="c", subcore_axis_name="s")

  @functools.partial(
      pl.kernel,
      mesh=mesh,
      out_type=jax.ShapeDtypeStruct((P * N, F), jnp.float32),
      scratch_types=[
          pltpu.VMEM_SHARED((N, F), jnp.float32),
          pltpu.VMEM((48, F), jnp.float32),
      ]
      + [pltpu.VMEM((_W,), jnp.int32) for _ in range(4)]
      + [pltpu.VMEM((_W,), jnp.int32) for _ in range(4)]
      + [pltpu.VMEM((_W,), jnp.float32) for _ in range(4)]
      + [pltpu.VMEM((_W,), jnp.int32) for _ in range(2)]
      + [pltpu.VMEM((_W, F), jnp.float32) for _ in range(2)]
      + [pltpu.SemaphoreType.DMA for _ in range(4)]
      + [pltpu.SemaphoreType.DMA for _ in range(2)]
      + [pltpu.SemaphoreType.DMA for _ in range(2)],
  )
  def spmm(x_hbm, src_hbm, dst_hbm, ew_hbm, out_hbm,
           y_sh, z2, s0, s1, s2, s3, d0, d1, d2, d3, e0, e1, e2, e3,
           gi0, gi1, r0_, r1_, es0, es1, es2, es3, gs0, gs1, ss0, ss1):
    srcb = (s0, s1, s2, s3)
    dstb = (d0, d1, d2, d3)
    ewb = (e0, e1, e2, e3)
    gidxb = (gi0, gi1)
    rowsb = (r0_, r1_)
    esem = (es0, es1, es2, es3)
    gsem = (gs0, gs1)
    ssem = (ss0, ss1)
    c = lax.axis_index("c")
    s = lax.axis_index("s")
    zero16 = jnp.zeros((_LANES,), jnp.float32)

    def _fill2(i, _):
      r = i // (F // _LANES)
      col = (i % (F // _LANES)) * _LANES
      z2[r, pl.ds(col, _LANES)] = zero16
      return 0
    lax.fori_loop(0, 48 * (F // _LANES), _fill2, 0)

    n_iters = (nwin + nsub - 1) // nsub

    for k in range(p_per_core):
      p = c * p_per_core + k
      r0 = s * rpt
      for q in range(rpt // 48):
        pltpu.sync_copy(z2, y_sh.at[pl.ds(r0 + q * 48, 48)])
      @pl.when(s == nsub - 1)
      def _zero_rem():
        pltpu.sync_copy(z2.at[pl.ds(0, rem)], y_sh.at[pl.ds(nsub * rpt, rem)])
      plsc.subcore_barrier()

      base = p * N

      def _guard(it):
        return jnp.logical_and(it >= 0, it * nsub + s < nwin)

      def _edges(it, b4):
        @pl.when(_guard(it))
        def _():
          e_off = (it * nsub + s) * _W
          pltpu.async_copy(src_hbm.at[pl.ds(e_off, _W)], srcb[b4], esem[b4])
          pltpu.async_copy(dst_hbm.at[pl.ds(e_off, _W)], dstb[b4], esem[b4])
          pltpu.async_copy(ew_hbm.at[pl.ds(e_off, _W)], ewb[b4], esem[b4])

      def _launch(it, b4, b2):
        @pl.when(_guard(it))
        def _():
          e_off = (it * nsub + s) * _W
          pltpu.make_async_copy(src_hbm.at[pl.ds(e_off, _W)], srcb[b4],
                                esem[b4]).wait()
          pltpu.make_async_copy(dst_hbm.at[pl.ds(e_off, _W)], dstb[b4],
                                esem[b4]).wait()
          pltpu.make_async_copy(ew_hbm.at[pl.ds(e_off, _W)], ewb[b4],
                                esem[b4]).wait()
          for j in range(_W // _LANES):
            sl = pl.ds(j * _LANES, _LANES)
            gidxb[b2][sl] = srcb[b4][sl] + base
          pltpu.async_copy(x_hbm.at[gidxb[b2]], rowsb[b2], gsem[b2])

      def _process(it, b4, b2):
        @pl.when(_guard(it))
        def _():
          pltpu.make_async_copy(x_hbm.at[gidxb[b2]], rowsb[b2],
                                gsem[b2]).wait()

          def _scale(g, _):
            nv = ewb[b4][pl.ds(g * _LANES, _LANES)]
            for l in range(_LANES):
              r = g * _LANES + l
              nb = jnp.full((_LANES,), nv[l])
              for j in range(F // _LANES):
                sl = pl.ds(j * _LANES, _LANES)
                rowsb[b2][r, sl] = rowsb[b2][r, sl] * nb
            return 0
          lax.fori_loop(0, _W // _LANES, _scale, 0)
          pltpu.async_copy(rowsb[b2], y_sh.at[dstb[b4]], ssem[b2], add=True)

      def _drain(it, b4, b2):
        @pl.when(_guard(it))
        def _():
          pltpu.make_async_copy(rowsb[b2], y_sh.at[dstb[b4]], ssem[b2]).wait()

      _edges(0, 0)

      def _quad_body(i, _):
        for q in range(4):
          it = i * 4 + q
          _drain(it - 2, (q - 2) % 4, q % 2)
          _edges(it + 1, (q + 1) % 4)
          _launch(it, q, q % 2)
          _process(it - 1, (q - 1) % 4, (q - 1) % 2)
        return 0
      lax.fori_loop(0, (n_iters + 4) // 4 + 1, _quad_body, 0)
      plsc.subcore_barrier()
      pltpu.sync_copy(y_sh.at[pl.ds(r0, rpt)],
                      out_hbm.at[pl.ds(p * N + r0, rpt)])
      @pl.when(s == nsub - 1)
      def _copy_rem():
        pltpu.sync_copy(y_sh.at[pl.ds(nsub * rpt, rem)],
                        out_hbm.at[pl.ds(p * N + nsub * rpt, rem)])
      plsc.subcore_barrier()

  return spmm


def _prep_body(deg_ref, x_ref, xs_ref, dinv_ref):
  deg = deg_ref[...]
  dinv = jnp.where(deg > 0.0, lax.rsqrt(jnp.where(deg > 0.0, deg, 1.0)), 0.0)
  xs_ref[...] = x_ref[...] * dinv
  dinv_ref[...] = dinv


def _gate_body(y_ref, dinv_ref, wz_ref, wh_ref, wlz_ref, wlh_ref,
               bz_ref, bh_ref, blz_ref, blh_ref, att_ref, out_ref):
  P = y_ref.shape[0]
  att = att_ref[...]
  att = att - jnp.max(att, axis=1, keepdims=True)
  ea = jnp.exp(att)
  probs = ea / jnp.sum(ea, axis=1, keepdims=True)
  az = jnp.dot(wz_ref[...], wlz_ref[...], preferred_element_type=jnp.float32)
  ah = jnp.dot(wh_ref[...], wlh_ref[...], preferred_element_type=jnp.float32)
  cz = jnp.dot(bz_ref[...], wlz_ref[...],
               preferred_element_type=jnp.float32) + blz_ref[...]
  ch = jnp.dot(bh_ref[...], wlh_ref[...],
               preferred_element_type=jnp.float32) + blh_ref[...]
  dinv = dinv_ref[...]
  acc = jnp.zeros(out_ref.shape, jnp.float32)
  for p in range(P):
    yp = y_ref[p] * dinv
    z = jax.nn.sigmoid(jnp.dot(yp, az, preferred_element_type=jnp.float32)
                       + cz)
    t = jnp.tanh(jnp.dot(yp, ah, preferred_element_type=jnp.float32) + ch)
    acc = acc + probs[0, p] * (1.0 - z) * t
  out_ref[...] = acc


def kernel(x, edge_index, edge_attr, Wz, bz, Wr, br, Wh, bh,
           Wlz, blz, Wlr, blr, Wlh, blh, attention):
  N, F, P = x.shape
  E = edge_index.shape[1]
  OUT = Wz.shape[1]
  del Wr, br, Wlr, blr

  xT = jnp.transpose(x, (2, 0, 1)).reshape(P * N, F)
  src = edge_index[0]
  dst = edge_index[1]

  deg = _make_sc_deg(N, E)(dst, edge_attr)
  deg2 = deg[:N].reshape(N, 1)

  R1 = 400
  xs, dinv = pl.pallas_call(
      _prep_body,
      grid=(P, N // R1),
      in_specs=[
          pl.BlockSpec((R1, 1), lambda p, i: (i, 0)),
          pl.BlockSpec((R1, F), lambda p, i, n_blocks=N // R1:
                       (p * n_blocks + i, 0)),
      ],
      out_specs=[
          pl.BlockSpec((R1, F), lambda p, i, n_blocks=N // R1:
                       (p * n_blocks + i, 0)),
          pl.BlockSpec((R1, 1), lambda p, i: (i, 0)),
      ],
      out_shape=[
          jax.ShapeDtypeStruct((P * N, F), jnp.float32),
          jax.ShapeDtypeStruct((N, 1), jnp.float32),
      ],
  )(deg2, xT)

  y = _make_sc_spmm(N, E, F, P)(xs, src, dst, edge_attr)
  y = y.reshape(P, N, F)

  R = 1000
  out = pl.pallas_call(
      _gate_body,
      grid=(N // R,),
      in_specs=[
          pl.BlockSpec((P, R, F), lambda i: (0, i, 0)),
          pl.BlockSpec((R, 1), lambda i: (i, 0)),
          pl.BlockSpec((F, OUT), lambda i: (0, 0)),
          pl.BlockSpec((F, OUT), lambda i: (0, 0)),
          pl.BlockSpec((OUT, OUT), lambda i: (0, 0)),
          pl.BlockSpec((OUT, OUT), lambda i: (0, 0)),
          pl.BlockSpec((1, OUT), lambda i: (0, 0)),
          pl.BlockSpec((1, OUT), lambda i: (0, 0)),
          pl.BlockSpec((1, OUT), lambda i: (0, 0)),
          pl.BlockSpec((1, OUT), lambda i: (0, 0)),
          pl.BlockSpec((1, P), lambda i: (0, 0)),
      ],
      out_specs=pl.BlockSpec((R, OUT), lambda i: (i, 0)),
      out_shape=jax.ShapeDtypeStruct((N, OUT), jnp.float32),
  )(y, dinv, Wz, Wh, Wlz, Wlh,
    bz.reshape(1, OUT), bh.reshape(1, OUT),
    blz.reshape(1, OUT), blh.reshape(1, OUT),
    attention.reshape(1, P))
  return out


# speedup vs baseline: 34.3228x; 1.0599x over previous
import functools

import jax
import jax.numpy as jnp
from jax import lax
from jax.experimental import pallas as pl
from jax.experimental.pallas import tpu as pltpu
from jax.experimental.pallas import tpu_sc as plsc


_LANES = 16
_W = 128


def _make_sc_deg(N, E):
  assert E % _W == 0 and N % 8 == 0
  nwin = E // _W
  info = plsc.get_sparse_core_info()
  nsub = info.num_subcores
  npad = ((N + 2047) // 2048) * 2048
  zchunk = 2048
  mesh = plsc.VectorSubcoreMesh(core_axis_name="c", subcore_axis_name="s")

  @functools.partial(
      pl.kernel,
      mesh=mesh,
      out_type=jax.ShapeDtypeStruct((npad,), jnp.float32),
      scratch_types=(
          [pltpu.VMEM_SHARED((npad,), jnp.float32)]
          + [pltpu.VMEM((zchunk,), jnp.float32)]
          + [pltpu.VMEM((_W,), jnp.int32) for _ in range(4)]
          + [pltpu.VMEM((_W,), jnp.float32) for _ in range(4)]
          + [pltpu.SemaphoreType.DMA for _ in range(4)]
          + [pltpu.SemaphoreType.DMA for _ in range(2)]
      ),
  )
  def deg_kernel(dst_hbm, ew_hbm, out_hbm, deg_sh, z1,
                 d0, d1, d2, d3, e0, e1, e2, e3,
                 es0, es1, es2, es3, ss0, ss1):
    dstb = (d0, d1, d2, d3)
    ewb = (e0, e1, e2, e3)
    esem = (es0, es1, es2, es3)
    ssem = (ss0, ss1)
    c = lax.axis_index("c")
    s = lax.axis_index("s")
    zero16 = jnp.zeros((_LANES,), jnp.float32)

    def _fill1(i, _):
      z1[pl.ds(i * _LANES, _LANES)] = zero16
      return 0
    lax.fori_loop(0, zchunk // _LANES, _fill1, 0)

    @pl.when(s == 0)
    def _zero_deg():
      for k in range(npad // zchunk):
        pltpu.sync_copy(z1, deg_sh.at[pl.ds(k * zchunk, zchunk)])
    plsc.subcore_barrier()

    n_iters = (nwin + nsub - 1) // nsub

    def _guard(it):
      return jnp.logical_and(it >= 0, it * nsub + s < nwin)

    def _edges(it, b4):
      @pl.when(_guard(it))
      def _():
        e_off = (it * nsub + s) * _W
        pltpu.async_copy(dst_hbm.at[pl.ds(e_off, _W)], dstb[b4], esem[b4])
        pltpu.async_copy(ew_hbm.at[pl.ds(e_off, _W)], ewb[b4], esem[b4])

    def _launch(it, b4, b2):
      @pl.when(_guard(it))
      def _():
        e_off = (it * nsub + s) * _W
        pltpu.make_async_copy(dst_hbm.at[pl.ds(e_off, _W)], dstb[b4],
                              esem[b4]).wait()
        pltpu.make_async_copy(ew_hbm.at[pl.ds(e_off, _W)], ewb[b4],
                              esem[b4]).wait()
        pltpu.async_copy(ewb[b4], deg_sh.at[dstb[b4]], ssem[b2], add=True)

    def _drain(it, b4, b2):
      @pl.when(_guard(it))
      def _():
        pltpu.make_async_copy(ewb[b4], deg_sh.at[dstb[b4]], ssem[b2]).wait()

    _edges(0, 0)

    def _quad_body(i, _):
      for q in range(4):
        it = i * 4 + q
        _drain(it - 2, (q - 2) % 4, q % 2)
        _edges(it + 1, (q + 1) % 4)
        _launch(it, q, q % 2)
      return 0
    lax.fori_loop(0, (n_iters + 4) // 4 + 1, _quad_body, 0)
    plsc.subcore_barrier()

    chunk = npad // nsub
    @pl.when(c == 0)
    def _copy_out():
      pltpu.sync_copy(deg_sh.at[pl.ds(s * chunk, chunk)],
                      out_hbm.at[pl.ds(s * chunk, chunk)])

  return deg_kernel


def _make_sc_spmm(N, E, F, P):
  assert F % _LANES == 0 and E % _W == 0 and N % 8 == 0
  nwin = E // _W
  info = plsc.get_sparse_core_info()
  ncores, nsub = info.num_cores, info.num_subcores
  assert P % ncores == 0
  p_per_core = P // ncores
  rpt = (N // nsub) // 8 * 8
  rem = N - nsub * rpt
  mesh = plsc.VectorSubcoreMesh(core_axis_name="c", subcore_axis_name="s")

  @functools.partial(
      pl.kernel,
      mesh=mesh,
      out_type=jax.ShapeDtypeStruct((P * N, F), jnp.float32),
      scratch_types=[
          pltpu.VMEM_SHARED((N, F), jnp.float32),
          pltpu.VMEM((48, F), jnp.float32),
      ]
      + [pltpu.VMEM((_W,), jnp.int32) for _ in range(4)]
      + [pltpu.VMEM((_W,), jnp.int32) for _ in range(4)]
      + [pltpu.VMEM((_W,), jnp.float32) for _ in range(4)]
      + [pltpu.VMEM((_W,), jnp.int32) for _ in range(2)]
      + [pltpu.VMEM((_W, F), jnp.float32) for _ in range(2)]
      + [pltpu.VMEM((_W,), jnp.float32) for _ in range(2)]
      + [pltpu.SemaphoreType.DMA for _ in range(4)]
      + [pltpu.SemaphoreType.DMA for _ in range(2)]
      + [pltpu.SemaphoreType.DMA for _ in range(2)],
  )
  def spmm(x_hbm, src_hbm, dst_hbm, ew_hbm, dinv_hbm, out_hbm,
           y_sh, z2, s0, s1, s2, s3, d0, d1, d2, d3, e0, e1, e2, e3,
           gi0, gi1, r0_, r1_, dv0, dv1,
           es0, es1, es2, es3, gs0, gs1, ss0, ss1):
    dinvv = (dv0, dv1)
    srcb = (s0, s1, s2, s3)
    dstb = (d0, d1, d2, d3)
    ewb = (e0, e1, e2, e3)
    gidxb = (gi0, gi1)
    rowsb = (r0_, r1_)
    esem = (es0, es1, es2, es3)
    gsem = (gs0, gs1)
    ssem = (ss0, ss1)
    c = lax.axis_index("c")
    s = lax.axis_index("s")
    zero16 = jnp.zeros((_LANES,), jnp.float32)

    def _fill2(i, _):
      r = i // (F // _LANES)
      col = (i % (F // _LANES)) * _LANES
      z2[r, pl.ds(col, _LANES)] = zero16
      return 0
    lax.fori_loop(0, 48 * (F // _LANES), _fill2, 0)

    n_iters = (nwin + nsub - 1) // nsub

    for k in range(p_per_core):
      p = c * p_per_core + k
      r0 = s * rpt
      for q in range(rpt // 48):
        pltpu.sync_copy(z2, y_sh.at[pl.ds(r0 + q * 48, 48)])
      @pl.when(s == nsub - 1)
      def _zero_rem():
        pltpu.sync_copy(z2.at[pl.ds(0, rem)], y_sh.at[pl.ds(nsub * rpt, rem)])
      plsc.subcore_barrier()

      base = p * N

      def _guard(it):
        return jnp.logical_and(it >= 0, it * nsub + s < nwin)

      def _edges(it, b4):
        @pl.when(_guard(it))
        def _():
          e_off = (it * nsub + s) * _W
          pltpu.async_copy(src_hbm.at[pl.ds(e_off, _W)], srcb[b4], esem[b4])
          pltpu.async_copy(dst_hbm.at[pl.ds(e_off, _W)], dstb[b4], esem[b4])
          pltpu.async_copy(ew_hbm.at[pl.ds(e_off, _W)], ewb[b4], esem[b4])

      def _launch(it, b4, b2):
        @pl.when(_guard(it))
        def _():
          e_off = (it * nsub + s) * _W
          pltpu.make_async_copy(src_hbm.at[pl.ds(e_off, _W)], srcb[b4],
                                esem[b4]).wait()
          pltpu.make_async_copy(dst_hbm.at[pl.ds(e_off, _W)], dstb[b4],
                                esem[b4]).wait()
          pltpu.make_async_copy(ew_hbm.at[pl.ds(e_off, _W)], ewb[b4],
                                esem[b4]).wait()
          for j in range(_W // _LANES):
            sl = pl.ds(j * _LANES, _LANES)
            gidxb[b2][sl] = srcb[b4][sl] + base
          pltpu.async_copy(x_hbm.at[gidxb[b2]], rowsb[b2], gsem[b2])
          pltpu.async_copy(dinv_hbm.at[srcb[b4]], dinvv[b2], gsem[b2])

      def _process(it, b4, b2):
        @pl.when(_guard(it))
        def _():
          pltpu.make_async_copy(x_hbm.at[gidxb[b2]], rowsb[b2],
                                gsem[b2]).wait()
          pltpu.make_async_copy(dinv_hbm.at[srcb[b4]], dinvv[b2],
                                gsem[b2]).wait()

          def _scale(g, _):
            sl0 = pl.ds(g * _LANES, _LANES)
            nv = dinvv[b2][sl0] * ewb[b4][sl0]
            for l in range(_LANES):
              r = g * _LANES + l
              nb = jnp.full((_LANES,), nv[l])
              for j in range(F // _LANES):
                sl = pl.ds(j * _LANES, _LANES)
                rowsb[b2][r, sl] = rowsb[b2][r, sl] * nb
            return 0
          lax.fori_loop(0, _W // _LANES, _scale, 0)
          pltpu.async_copy(rowsb[b2], y_sh.at[dstb[b4]], ssem[b2], add=True)

      def _drain(it, b4, b2):
        @pl.when(_guard(it))
        def _():
          pltpu.make_async_copy(rowsb[b2], y_sh.at[dstb[b4]], ssem[b2]).wait()

      _edges(0, 0)

      def _quad_body(i, _):
        for q in range(4):
          it = i * 4 + q
          _drain(it - 2, (q - 2) % 4, q % 2)
          _edges(it + 1, (q + 1) % 4)
          _launch(it, q, q % 2)
          _process(it - 1, (q - 1) % 4, (q - 1) % 2)
        return 0
      lax.fori_loop(0, (n_iters + 4) // 4 + 1, _quad_body, 0)
      plsc.subcore_barrier()
      pltpu.sync_copy(y_sh.at[pl.ds(r0, rpt)],
                      out_hbm.at[pl.ds(p * N + r0, rpt)])
      @pl.when(s == nsub - 1)
      def _copy_rem():
        pltpu.sync_copy(y_sh.at[pl.ds(nsub * rpt, rem)],
                        out_hbm.at[pl.ds(p * N + nsub * rpt, rem)])
      plsc.subcore_barrier()

  return spmm


def _prep_body(deg_ref, dinv_ref):
  deg = deg_ref[...]
  dinv_ref[...] = jnp.where(
      deg > 0.0, lax.rsqrt(jnp.where(deg > 0.0, deg, 1.0)), 0.0)


def _gate_body(y_ref, dinv_ref, wz_ref, wh_ref, wlz_ref, wlh_ref,
               bz_ref, bh_ref, blz_ref, blh_ref, att_ref, out_ref):
  P = y_ref.shape[0]
  att = att_ref[...]
  att = att - jnp.max(att, axis=1, keepdims=True)
  ea = jnp.exp(att)
  probs = ea / jnp.sum(ea, axis=1, keepdims=True)
  az = jnp.dot(wz_ref[...], wlz_ref[...], preferred_element_type=jnp.float32)
  ah = jnp.dot(wh_ref[...], wlh_ref[...], preferred_element_type=jnp.float32)
  cz = jnp.dot(bz_ref[...], wlz_ref[...],
               preferred_element_type=jnp.float32) + blz_ref[...]
  ch = jnp.dot(bh_ref[...], wlh_ref[...],
               preferred_element_type=jnp.float32) + blh_ref[...]
  dinv = dinv_ref[...]
  acc = jnp.zeros(out_ref.shape, jnp.float32)
  for p in range(P):
    yp = y_ref[p] * dinv
    z = jax.nn.sigmoid(jnp.dot(yp, az, preferred_element_type=jnp.float32)
                       + cz)
    t = jnp.tanh(jnp.dot(yp, ah, preferred_element_type=jnp.float32) + ch)
    acc = acc + probs[0, p] * (1.0 - z) * t
  out_ref[...] = acc


def kernel(x, edge_index, edge_attr, Wz, bz, Wr, br, Wh, bh,
           Wlz, blz, Wlr, blr, Wlh, blh, attention):
  N, F, P = x.shape
  E = edge_index.shape[1]
  OUT = Wz.shape[1]
  del Wr, br, Wlr, blr

  xT = jnp.transpose(x, (2, 0, 1)).reshape(P * N, F)
  src = edge_index[0]
  dst = edge_index[1]

  deg = _make_sc_deg(N, E)(dst, edge_attr)
  npad = deg.shape[0]
  deg2 = deg.reshape(npad // 128, 128)

  dinv_pad = pl.pallas_call(
      _prep_body,
      in_specs=[pl.BlockSpec(deg2.shape, lambda: (0, 0))],
      out_specs=pl.BlockSpec(deg2.shape, lambda: (0, 0)),
      out_shape=jax.ShapeDtypeStruct(deg2.shape, jnp.float32),
  )(deg2)
  dinv_flat = dinv_pad.reshape(npad)
  dinv = dinv_flat[:N].reshape(N, 1)

  y = _make_sc_spmm(N, E, F, P)(xT, src, dst, edge_attr, dinv_flat)
  y = y.reshape(P, N, F)

  R = 1000
  out = pl.pallas_call(
      _gate_body,
      grid=(N // R,),
      in_specs=[
          pl.BlockSpec((P, R, F), lambda i: (0, i, 0)),
          pl.BlockSpec((R, 1), lambda i: (i, 0)),
          pl.BlockSpec((F, OUT), lambda i: (0, 0)),
          pl.BlockSpec((F, OUT), lambda i: (0, 0)),
          pl.BlockSpec((OUT, OUT), lambda i: (0, 0)),
          pl.BlockSpec((OUT, OUT), lambda i: (0, 0)),
          pl.BlockSpec((1, OUT), lambda i: (0, 0)),
          pl.BlockSpec((1, OUT), lambda i: (0, 0)),
          pl.BlockSpec((1, OUT), lambda i: (0, 0)),
          pl.BlockSpec((1, OUT), lambda i: (0, 0)),
          pl.BlockSpec((1, P), lambda i: (0, 0)),
      ],
      out_specs=pl.BlockSpec((R, OUT), lambda i: (i, 0)),
      out_shape=jax.ShapeDtypeStruct((N, OUT), jnp.float32),
  )(y, dinv, Wz, Wh, Wlz, Wlh,
    bz.reshape(1, OUT), bh.reshape(1, OUT),
    blz.reshape(1, OUT), blh.reshape(1, OUT),
    attention.reshape(1, P))
  return out


# speedup vs baseline: 34.7513x; 1.0125x over previous
import functools

import jax
import jax.numpy as jnp
from jax import lax
from jax.experimental import pallas as pl
from jax.experimental.pallas import tpu as pltpu
from jax.experimental.pallas import tpu_sc as plsc


_LANES = 16
_W = 128


def _rsqrt_newton(d):
  i = lax.bitcast_convert_type(d, jnp.int32)
  y = lax.bitcast_convert_type(
      jnp.int32(0x5F3759DF) - lax.shift_right_logical(i, 1), jnp.float32)
  half = d * 0.5
  for _ in range(4):
    y = y * (1.5 - half * y * y)
  return jnp.where(d > 0.0, y, 0.0)


def _make_sc_deg(N, E):
  assert E % _W == 0 and N % 8 == 0
  nwin = E // _W
  info = plsc.get_sparse_core_info()
  nsub = info.num_subcores
  npad = ((N + 2047) // 2048) * 2048
  zchunk = 2048
  mesh = plsc.VectorSubcoreMesh(core_axis_name="c", subcore_axis_name="s")

  @functools.partial(
      pl.kernel,
      mesh=mesh,
      out_type=jax.ShapeDtypeStruct((npad,), jnp.float32),
      scratch_types=(
          [pltpu.VMEM_SHARED((npad,), jnp.float32)]
          + [pltpu.VMEM((zchunk,), jnp.float32)]
          + [pltpu.VMEM((_W,), jnp.int32) for _ in range(4)]
          + [pltpu.VMEM((_W,), jnp.float32) for _ in range(4)]
          + [pltpu.SemaphoreType.DMA for _ in range(4)]
          + [pltpu.SemaphoreType.DMA for _ in range(2)]
      ),
  )
  def deg_kernel(dst_hbm, ew_hbm, out_hbm, deg_sh, z1,
                 d0, d1, d2, d3, e0, e1, e2, e3,
                 es0, es1, es2, es3, ss0, ss1):
    dstb = (d0, d1, d2, d3)
    ewb = (e0, e1, e2, e3)
    esem = (es0, es1, es2, es3)
    ssem = (ss0, ss1)
    c = lax.axis_index("c")
    s = lax.axis_index("s")
    zero16 = jnp.zeros((_LANES,), jnp.float32)

    def _fill1(i, _):
      z1[pl.ds(i * _LANES, _LANES)] = zero16
      return 0
    lax.fori_loop(0, zchunk // _LANES, _fill1, 0)

    @pl.when(s == 0)
    def _zero_deg():
      for k in range(npad // zchunk):
        pltpu.sync_copy(z1, deg_sh.at[pl.ds(k * zchunk, zchunk)])
    plsc.subcore_barrier()

    n_iters = (nwin + nsub - 1) // nsub

    def _guard(it):
      return jnp.logical_and(it >= 0, it * nsub + s < nwin)

    def _edges(it, b4):
      @pl.when(_guard(it))
      def _():
        e_off = (it * nsub + s) * _W
        pltpu.async_copy(dst_hbm.at[pl.ds(e_off, _W)], dstb[b4], esem[b4])
        pltpu.async_copy(ew_hbm.at[pl.ds(e_off, _W)], ewb[b4], esem[b4])

    def _launch(it, b4, b2):
      @pl.when(_guard(it))
      def _():
        e_off = (it * nsub + s) * _W
        pltpu.make_async_copy(dst_hbm.at[pl.ds(e_off, _W)], dstb[b4],
                              esem[b4]).wait()
        pltpu.make_async_copy(ew_hbm.at[pl.ds(e_off, _W)], ewb[b4],
                              esem[b4]).wait()
        pltpu.async_copy(ewb[b4], deg_sh.at[dstb[b4]], ssem[b2], add=True)

    def _drain(it, b4, b2):
      @pl.when(_guard(it))
      def _():
        pltpu.make_async_copy(ewb[b4], deg_sh.at[dstb[b4]], ssem[b2]).wait()

    _edges(0, 0)

    def _quad_body(i, _):
      for q in range(4):
        it = i * 4 + q
        _drain(it - 2, (q - 2) % 4, q % 2)
        _edges(it + 1, (q + 1) % 4)
        _launch(it, q, q % 2)
      return 0
    lax.fori_loop(0, (n_iters + 4) // 4 + 1, _quad_body, 0)
    plsc.subcore_barrier()

    chunk = npad // nsub
    @pl.when(c == 0)
    def _copy_out():
      pltpu.sync_copy(deg_sh.at[pl.ds(s * chunk, chunk)],
                      out_hbm.at[pl.ds(s * chunk, chunk)])

  return deg_kernel


def _make_sc_spmm(N, E, F, P):
  assert F % _LANES == 0 and E % _W == 0 and N % 8 == 0
  nwin = E // _W
  info = plsc.get_sparse_core_info()
  ncores, nsub = info.num_cores, info.num_subcores
  assert P % ncores == 0
  p_per_core = P // ncores
  rpt = (N // nsub) // 8 * 8
  rem = N - nsub * rpt
  mesh = plsc.VectorSubcoreMesh(core_axis_name="c", subcore_axis_name="s")

  npad = ((N + 2047) // 2048) * 2048
  zchunk = 2048
  chunk = npad // nsub

  @functools.partial(
      pl.kernel,
      mesh=mesh,
      out_type=[
          jax.ShapeDtypeStruct((ncores * npad,), jnp.float32),
          jax.ShapeDtypeStruct((P * N, F), jnp.float32),
      ],
      scratch_types=[
          pltpu.VMEM_SHARED((N, F), jnp.float32),
          pltpu.VMEM_SHARED((npad,), jnp.float32),
          pltpu.VMEM((48, F), jnp.float32),
          pltpu.VMEM((zchunk,), jnp.float32),
          pltpu.VMEM((chunk,), jnp.float32),
          pltpu.VMEM((chunk,), jnp.float32),
      ]
      + [pltpu.VMEM((_W,), jnp.int32) for _ in range(4)]
      + [pltpu.VMEM((_W,), jnp.int32) for _ in range(4)]
      + [pltpu.VMEM((_W,), jnp.float32) for _ in range(4)]
      + [pltpu.VMEM((_W,), jnp.int32) for _ in range(2)]
      + [pltpu.VMEM((_W,), jnp.int32) for _ in range(2)]
      + [pltpu.VMEM((_W, F), jnp.float32) for _ in range(2)]
      + [pltpu.VMEM((_W,), jnp.float32) for _ in range(2)]
      + [pltpu.SemaphoreType.DMA for _ in range(4)]
      + [pltpu.SemaphoreType.DMA for _ in range(2)]
      + [pltpu.SemaphoreType.DMA for _ in range(2)],
  )
  def spmm(x_hbm, src_hbm, dst_hbm, ew_hbm, dinv_hbm, out_hbm,
           y_sh, deg_sh, z2, z1, degl, dinvl,
           s0, s1, s2, s3, d0, d1, d2, d3, e0, e1, e2, e3,
           gi0, gi1, di0, di1, r0_, r1_, dv0, dv1,
           es0, es1, es2, es3, gs0, gs1, ss0, ss1):
    didxb = (di0, di1)
    dinvv = (dv0, dv1)
    srcb = (s0, s1, s2, s3)
    dstb = (d0, d1, d2, d3)
    ewb = (e0, e1, e2, e3)
    gidxb = (gi0, gi1)
    rowsb = (r0_, r1_)
    esem = (es0, es1, es2, es3)
    gsem = (gs0, gs1)
    ssem = (ss0, ss1)
    c = lax.axis_index("c")
    s = lax.axis_index("s")
    zero16 = jnp.zeros((_LANES,), jnp.float32)

    def _fill2(i, _):
      r = i // (F // _LANES)
      col = (i % (F // _LANES)) * _LANES
      z2[r, pl.ds(col, _LANES)] = zero16
      return 0
    lax.fori_loop(0, 48 * (F // _LANES), _fill2, 0)

    n_iters = (nwin + nsub - 1) // nsub

    def _fill1(i, _):
      z1[pl.ds(i * _LANES, _LANES)] = zero16
      return 0
    lax.fori_loop(0, zchunk // _LANES, _fill1, 0)

    @pl.when(s == 0)
    def _zero_deg():
      for kk in range(npad // zchunk):
        pltpu.sync_copy(z1, deg_sh.at[pl.ds(kk * zchunk, zchunk)])
    plsc.subcore_barrier()

    def _dguard(it):
      return jnp.logical_and(it >= 0, it * nsub + s < nwin)

    def _dedges(it, b4):
      @pl.when(_dguard(it))
      def _():
        e_off = (it * nsub + s) * _W
        pltpu.async_copy(dst_hbm.at[pl.ds(e_off, _W)], dstb[b4], esem[b4])
        pltpu.async_copy(ew_hbm.at[pl.ds(e_off, _W)], ewb[b4], esem[b4])

    def _dlaunch(it, b4, b2):
      @pl.when(_dguard(it))
      def _():
        e_off = (it * nsub + s) * _W
        pltpu.make_async_copy(dst_hbm.at[pl.ds(e_off, _W)], dstb[b4],
                              esem[b4]).wait()
        pltpu.make_async_copy(ew_hbm.at[pl.ds(e_off, _W)], ewb[b4],
                              esem[b4]).wait()
        pltpu.async_copy(ewb[b4], deg_sh.at[dstb[b4]], ssem[b2], add=True)

    def _ddrain(it, b4, b2):
      @pl.when(_dguard(it))
      def _():
        pltpu.make_async_copy(ewb[b4], deg_sh.at[dstb[b4]], ssem[b2]).wait()

    _dedges(0, 0)

    def _dquad(i, _):
      for q in range(4):
        it = i * 4 + q
        _ddrain(it - 2, (q - 2) % 4, q % 2)
        _dedges(it + 1, (q + 1) % 4)
        _dlaunch(it, q, q % 2)
      return 0
    lax.fori_loop(0, (n_iters + 4) // 4 + 1, _dquad, 0)
    plsc.subcore_barrier()

    pltpu.sync_copy(deg_sh.at[pl.ds(s * chunk, chunk)], degl)

    def _newton(i, _):
      sl = pl.ds(i * _LANES, _LANES)
      dinvl[sl] = _rsqrt_newton(degl[sl])
      return 0
    lax.fori_loop(0, chunk // _LANES, _newton, 0)
    pltpu.sync_copy(dinvl, dinv_hbm.at[pl.ds(c * npad + s * chunk, chunk)])
    plsc.subcore_barrier()
    cbase = c * npad

    for k in range(p_per_core):
      p = c * p_per_core + k
      r0 = s * rpt
      for q in range(rpt // 48):
        pltpu.sync_copy(z2, y_sh.at[pl.ds(r0 + q * 48, 48)])
      @pl.when(s == nsub - 1)
      def _zero_rem():
        pltpu.sync_copy(z2.at[pl.ds(0, rem)], y_sh.at[pl.ds(nsub * rpt, rem)])
      plsc.subcore_barrier()

      base = p * N

      def _guard(it):
        return jnp.logical_and(it >= 0, it * nsub + s < nwin)

      def _edges(it, b4):
        @pl.when(_guard(it))
        def _():
          e_off = (it * nsub + s) * _W
          pltpu.async_copy(src_hbm.at[pl.ds(e_off, _W)], srcb[b4], esem[b4])
          pltpu.async_copy(dst_hbm.at[pl.ds(e_off, _W)], dstb[b4], esem[b4])
          pltpu.async_copy(ew_hbm.at[pl.ds(e_off, _W)], ewb[b4], esem[b4])

      def _launch(it, b4, b2):
        @pl.when(_guard(it))
        def _():
          e_off = (it * nsub + s) * _W
          pltpu.make_async_copy(src_hbm.at[pl.ds(e_off, _W)], srcb[b4],
                                esem[b4]).wait()
          pltpu.make_async_copy(dst_hbm.at[pl.ds(e_off, _W)], dstb[b4],
                                esem[b4]).wait()
          pltpu.make_async_copy(ew_hbm.at[pl.ds(e_off, _W)], ewb[b4],
                                esem[b4]).wait()
          for j in range(_W // _LANES):
            sl = pl.ds(j * _LANES, _LANES)
            sv = srcb[b4][sl]
            gidxb[b2][sl] = sv + base
            didxb[b2][sl] = sv + cbase
          pltpu.async_copy(x_hbm.at[gidxb[b2]], rowsb[b2], gsem[b2])
          pltpu.async_copy(dinv_hbm.at[didxb[b2]], dinvv[b2], gsem[b2])

      def _process(it, b4, b2):
        @pl.when(_guard(it))
        def _():
          pltpu.make_async_copy(x_hbm.at[gidxb[b2]], rowsb[b2],
                                gsem[b2]).wait()
          pltpu.make_async_copy(dinv_hbm.at[didxb[b2]], dinvv[b2],
                                gsem[b2]).wait()

          def _scale(g, _):
            sl0 = pl.ds(g * _LANES, _LANES)
            nv = dinvv[b2][sl0] * ewb[b4][sl0]
            for l in range(_LANES):
              r = g * _LANES + l
              nb = jnp.full((_LANES,), nv[l])
              for j in range(F // _LANES):
                sl = pl.ds(j * _LANES, _LANES)
                rowsb[b2][r, sl] = rowsb[b2][r, sl] * nb
            return 0
          lax.fori_loop(0, _W // _LANES, _scale, 0)
          pltpu.async_copy(rowsb[b2], y_sh.at[dstb[b4]], ssem[b2], add=True)

      def _drain(it, b4, b2):
        @pl.when(_guard(it))
        def _():
          pltpu.make_async_copy(rowsb[b2], y_sh.at[dstb[b4]], ssem[b2]).wait()

      _edges(0, 0)

      def _quad_body(i, _):
        for q in range(4):
          it = i * 4 + q
          _drain(it - 2, (q - 2) % 4, q % 2)
          _edges(it + 1, (q + 1) % 4)
          _launch(it, q, q % 2)
          _process(it - 1, (q - 1) % 4, (q - 1) % 2)
        return 0
      lax.fori_loop(0, (n_iters + 4) // 4 + 1, _quad_body, 0)
      plsc.subcore_barrier()
      pltpu.sync_copy(y_sh.at[pl.ds(r0, rpt)],
                      out_hbm.at[pl.ds(p * N + r0, rpt)])
      @pl.when(s == nsub - 1)
      def _copy_rem():
        pltpu.sync_copy(y_sh.at[pl.ds(nsub * rpt, rem)],
                        out_hbm.at[pl.ds(p * N + nsub * rpt, rem)])
      plsc.subcore_barrier()

  return spmm


def _prep_body(deg_ref, dinv_ref):
  deg = deg_ref[...]
  dinv_ref[...] = jnp.where(
      deg > 0.0, lax.rsqrt(jnp.where(deg > 0.0, deg, 1.0)), 0.0)


def _gate_body(y_ref, dinv_ref, wz_ref, wh_ref, wlz_ref, wlh_ref,
               bz_ref, bh_ref, blz_ref, blh_ref, att_ref, out_ref):
  P = y_ref.shape[0]
  att = att_ref[...]
  att = att - jnp.max(att, axis=1, keepdims=True)
  ea = jnp.exp(att)
  probs = ea / jnp.sum(ea, axis=1, keepdims=True)
  az = jnp.dot(wz_ref[...], wlz_ref[...], preferred_element_type=jnp.float32)
  ah = jnp.dot(wh_ref[...], wlh_ref[...], preferred_element_type=jnp.float32)
  cz = jnp.dot(bz_ref[...], wlz_ref[...],
               preferred_element_type=jnp.float32) + blz_ref[...]
  ch = jnp.dot(bh_ref[...], wlh_ref[...],
               preferred_element_type=jnp.float32) + blh_ref[...]
  dinv = dinv_ref[...]
  acc = jnp.zeros(out_ref.shape, jnp.float32)
  for p in range(P):
    yp = y_ref[p] * dinv
    z = jax.nn.sigmoid(jnp.dot(yp, az, preferred_element_type=jnp.float32)
                       + cz)
    t = jnp.tanh(jnp.dot(yp, ah, preferred_element_type=jnp.float32) + ch)
    acc = acc + probs[0, p] * (1.0 - z) * t
  out_ref[...] = acc


def kernel(x, edge_index, edge_attr, Wz, bz, Wr, br, Wh, bh,
           Wlz, blz, Wlr, blr, Wlh, blh, attention):
  N, F, P = x.shape
  E = edge_index.shape[1]
  OUT = Wz.shape[1]
  del Wr, br, Wlr, blr

  xT = jnp.transpose(x, (2, 0, 1)).reshape(P * N, F)
  src = edge_index[0]
  dst = edge_index[1]

  dinv2, y = _make_sc_spmm(N, E, F, P)(xT, src, dst, edge_attr)
  dinv = dinv2[:N].reshape(N, 1)
  y = y.reshape(P, N, F)

  R = 1000
  out = pl.pallas_call(
      _gate_body,
      grid=(N // R,),
      in_specs=[
          pl.BlockSpec((P, R, F), lambda i: (0, i, 0)),
          pl.BlockSpec((R, 1), lambda i: (i, 0)),
          pl.BlockSpec((F, OUT), lambda i: (0, 0)),
          pl.BlockSpec((F, OUT), lambda i: (0, 0)),
          pl.BlockSpec((OUT, OUT), lambda i: (0, 0)),
          pl.BlockSpec((OUT, OUT), lambda i: (0, 0)),
          pl.BlockSpec((1, OUT), lambda i: (0, 0)),
          pl.BlockSpec((1, OUT), lambda i: (0, 0)),
          pl.BlockSpec((1, OUT), lambda i: (0, 0)),
          pl.BlockSpec((1, OUT), lambda i: (0, 0)),
          pl.BlockSpec((1, P), lambda i: (0, 0)),
      ],
      out_specs=pl.BlockSpec((R, OUT), lambda i: (i, 0)),
      out_shape=jax.ShapeDtypeStruct((N, OUT), jnp.float32),
  )(y, dinv, Wz, Wh, Wlz, Wlh,
    bz.reshape(1, OUT), bh.reshape(1, OUT),
    blz.reshape(1, OUT), blh.reshape(1, OUT),
    attention.reshape(1, P))
  return out
